# baseline (speedup 1.0000x reference)
import functools
import jax
import jax.numpy as jnp
from jax import lax
from jax.experimental import pallas as pl
from jax.experimental.pallas import tpu as pltpu
from jax.experimental.pallas import tpu_sc as plsc

TOP_K = 2
N_GROUP = 4
TOPK_GROUP = 2
ROUTED_SCALING = 2.5

NC = 2
NS = 16
NW = NC * NS


def _shift_right(x):
    return jnp.concatenate([jnp.zeros_like(x[:, :1]), x[:, :-1]], axis=1)


def _router_body(hidden_ref, gate_ref, bias_ref, pos_ref, w_ref, meta_ref,
                 *, T, E, B):
    h = hidden_ref[...]
    logits = jnp.dot(h, gate_ref[...], preferred_element_type=jnp.float32)
    scores = jax.nn.sigmoid(logits)
    s_choice = scores + bias_ref[...]
    sh1 = jnp.concatenate([s_choice[:, 1:], s_choice[:, :1]], axis=1)
    gs8 = s_choice + sh1
    lane_e = lax.broadcasted_iota(jnp.int32, (T, E), 1)
    even = (lane_e % 2) == 0
    gfull = jnp.where(even, gs8, -1e30)
    rank_g = jnp.zeros((T, E), jnp.int32)
    for hg in range(0, E, 2):
        sh = gfull[:, hg:hg + 1]
        rank_g += ((sh > gfull) | ((sh == gfull) & (hg < lane_e))).astype(jnp.int32)
    mask_even = jnp.where((rank_g < TOPK_GROUP) & even, 1.0, 0.0)
    mprev = jnp.concatenate([mask_even[:, -1:], mask_even[:, :-1]], axis=1)
    mask_e = jnp.where(even, mask_even, mprev)
    tmp = jnp.where(mask_e > 0, s_choice, 0.0)
    rank_e = jnp.zeros((T, E), jnp.int32)
    for he in range(E):
        sh = tmp[:, he:he + 1]
        rank_e += ((sh > tmp) | ((sh == tmp) & (he < lane_e))).astype(jnp.int32)
    sel = jnp.where(rank_e < TOP_K, 1.0, 0.0)
    w = sel * scores
    denom = jnp.sum(w, axis=1, keepdims=True) + 1e-20
    combine = (w / denom) * ROUTED_SCALING

    row_i = lax.broadcasted_iota(jnp.int32, (T, T), 0)
    col_i = lax.broadcasted_iota(jnp.int32, (T, T), 1)
    tri = jnp.where(row_i > col_i, 1.0, 0.0).astype(jnp.bfloat16)
    cnt_before = jnp.dot(tri, sel.astype(jnp.bfloat16),
                         preferred_element_type=jnp.float32)
    counts = jnp.sum(sel, axis=0, keepdims=True)
    cpad = jnp.ceil(counts / B) * B
    base = jnp.zeros_like(cpad)
    cur = cpad
    for _ in range(E - 1):
        cur = _shift_right(cur)
        base = base + cur
    lower = jnp.zeros_like(sel)
    cur = sel
    for _ in range(E - 1):
        cur = _shift_right(cur)
        lower = lower + cur
    m0 = sel * jnp.where(lower == 0, 1.0, 0.0)
    m1 = sel * jnp.where(lower == 1.0, 1.0, 0.0)
    posv = base + cnt_before
    pos0 = jnp.sum(m0 * posv, axis=1, keepdims=True)
    pos1 = jnp.sum(m1 * posv, axis=1, keepdims=True)
    w0 = jnp.sum(m0 * combine, axis=1, keepdims=True)
    w1 = jnp.sum(m1 * combine, axis=1, keepdims=True)
    lane0 = jnp.where(lane_e == 0, 1.0, 0.0)
    lane1 = jnp.where(lane_e == 1, 1.0, 0.0)
    pos_ref[...] = (pos0 * lane0 + pos1 * lane1).astype(jnp.int32)
    w_ref[...] = w0 * lane0 + w1 * lane1
    sub_i = lax.broadcasted_iota(jnp.int32, (E, E), 0)
    nb_row = jnp.where(sub_i == 0, 1.0, 0.0) * (cpad / B)
    bb_row = jnp.where(sub_i == 1, 1.0, 0.0) * (base / B)
    meta_ref[...] = (nb_row + bb_row).astype(jnp.int32)


def _invert_body(pos0r_ref, pos1r_ref, tok_ref, *, QB, T):
    qb = pl.program_id(0)
    qv = qb * QB + lax.broadcasted_iota(jnp.int32, (QB, 1), 0)
    p0 = pos0r_ref[...]
    p1 = pos1r_ref[...]
    trow = lax.broadcasted_iota(jnp.int32, (1, T), 1).astype(jnp.float32)
    eq0 = jnp.where(qv == p0, 1.0, 0.0)
    eq1 = jnp.where(qv == p1, 1.0, 0.0)
    tok = jnp.sum((eq0 + eq1) * trow, axis=1, keepdims=True)
    lane = lax.broadcasted_iota(jnp.int32, (QB, 8), 1)
    tok_ref[...] = jnp.where(lane >= 0, tok.astype(jnp.int32), 0)


def _sc_disperse(hidden_hbm, tok_hbm, sorted_h_hbm, rows_v, idx_v, sem,
                 *, PADTOT):
    wid = lax.axis_index("s") * NC + lax.axis_index("c")
    per_w = PADTOT // NW
    nchunk = per_w // 32
    for c in range(nchunk):
        q0 = wid * per_w + c * 32
        pltpu.sync_copy(tok_hbm.at[pl.ds(q0, 32)], idx_v)
        pltpu.async_copy(hidden_hbm.at[idx_v], rows_v, sem).wait()
        pltpu.sync_copy(rows_v, sorted_h_hbm.at[pl.ds(q0, 32)])


def _sc_collect(sorted_out_hbm, poscat_hbm, buf_hbm, rows_v, idx_v, sem, *, P):
    wid = lax.axis_index("s") * NC + lax.axis_index("c")
    per_w = P // NW
    nchunk = per_w // 32
    for c in range(nchunk):
        p0 = wid * per_w + c * 32
        pltpu.sync_copy(poscat_hbm.at[pl.ds(p0, 32)], idx_v)
        pltpu.async_copy(sorted_out_hbm.at[idx_v], rows_v, sem).wait()
        pltpu.sync_copy(rows_v, buf_hbm.at[pl.ds(p0, 32)])


def _grouped_body(nb_ref, bb_ref, sh_ref, wgu_ref, wd_ref, out_ref, *, I):
    e = pl.program_id(0)
    b = pl.program_id(1)

    @pl.when(b < nb_ref[e])
    def _compute():
        hc = sh_ref[...].astype(jnp.bfloat16)
        gu = jnp.dot(hc, wgu_ref[0], preferred_element_type=jnp.float32)
        g = gu[:, :I]
        u = gu[:, I:]
        act = (g * jax.nn.sigmoid(g) * u).astype(jnp.bfloat16)
        out_ref[...] = jnp.dot(act, wd_ref[0], preferred_element_type=jnp.float32)


def _final_body(hidden_ref, buf0_ref, buf1_ref, w_ref, sgu_ref, sd_ref, out_ref,
                *, I_sh):
    h = hidden_ref[...].astype(jnp.bfloat16)
    gu = jnp.dot(h, sgu_ref[...], preferred_element_type=jnp.float32)
    g = gu[:, :I_sh]
    u = gu[:, I_sh:]
    act = (g * jax.nn.sigmoid(g) * u).astype(jnp.bfloat16)
    sh = jnp.dot(act, sd_ref[...], preferred_element_type=jnp.float32)
    out_ref[...] = (w_ref[:, 0:1] * buf0_ref[...]
                    + w_ref[:, 1:2] * buf1_ref[...] + sh)


def kernel(hidden_states, gate_w, e_score_correction_bias, w_gate_up, w_down,
           shared_gate_up, shared_down):
    T, H = hidden_states.shape
    E = gate_w.shape[1]
    I = w_down.shape[1]
    I_sh = shared_down.shape[0]
    B = 256
    MB = T // B
    PADTOT = TOP_K * T + E * B
    P = TOP_K * T

    bias = e_score_correction_bias.reshape(1, E)
    wgu_bf = w_gate_up.astype(jnp.bfloat16)
    wd_bf = w_down.astype(jnp.bfloat16)
    sgu_bf = shared_gate_up.astype(jnp.bfloat16)
    sd_bf = shared_down.astype(jnp.bfloat16)

    pos_t8, w_t8, meta = pl.pallas_call(
        functools.partial(_router_body, T=T, E=E, B=B),
        in_specs=[
            pl.BlockSpec((T, H), lambda: (0, 0)),
            pl.BlockSpec((H, E), lambda: (0, 0)),
            pl.BlockSpec((1, E), lambda: (0, 0)),
        ],
        out_specs=(
            pl.BlockSpec((T, E), lambda: (0, 0)),
            pl.BlockSpec((T, E), lambda: (0, 0)),
            pl.BlockSpec((E, E), lambda: (0, 0)),
        ),
        out_shape=(
            jax.ShapeDtypeStruct((T, E), jnp.int32),
            jax.ShapeDtypeStruct((T, E), jnp.float32),
            jax.ShapeDtypeStruct((E, E), jnp.int32),
        ),
    )(hidden_states, gate_w, bias)

    pos0 = pos_t8[:, 0]
    pos1 = pos_t8[:, 1]
    poscat = jnp.concatenate([pos0, pos1])
    nb = meta[0]
    bb = meta[1]

    QB = 512
    tok8 = pl.pallas_call(
        functools.partial(_invert_body, QB=QB, T=T),
        grid=(PADTOT // QB,),
        in_specs=[
            pl.BlockSpec((1, T), lambda q: (0, 0)),
            pl.BlockSpec((1, T), lambda q: (0, 0)),
        ],
        out_specs=pl.BlockSpec((QB, E), lambda q: (q, 0)),
        out_shape=jax.ShapeDtypeStruct((PADTOT, E), jnp.int32),
    )(pos0.reshape(1, T), pos1.reshape(1, T))
    tok = tok8[:, 0]

    mesh = plsc.VectorSubcoreMesh(core_axis_name="c", subcore_axis_name="s")
    sorted_h = pl.kernel(
        functools.partial(_sc_disperse, PADTOT=PADTOT),
        mesh=mesh,
        out_type=jax.ShapeDtypeStruct((PADTOT, H), jnp.float32),
        scratch_types=[
            pltpu.VMEM((32, H), jnp.float32),
            pltpu.VMEM((32,), jnp.int32),
            pltpu.SemaphoreType.DMA,
        ],
    )(hidden_states, tok)

    nblk = PADTOT // B
    sorted_out = pl.pallas_call(
        functools.partial(_grouped_body, I=I),
        grid_spec=pltpu.PrefetchScalarGridSpec(
            num_scalar_prefetch=2,
            grid=(E, MB),
            in_specs=[
                pl.BlockSpec(
                    (B, H),
                    lambda e, b, nbr, bbr: (
                        jnp.minimum(bbr[e] + jnp.maximum(
                            jnp.minimum(b, nbr[e] - 1), 0), nblk - 1), 0)),
                pl.BlockSpec((1, H, 2 * I), lambda e, b, nbr, bbr: (e, 0, 0)),
                pl.BlockSpec((1, I, H), lambda e, b, nbr, bbr: (e, 0, 0)),
            ],
            out_specs=pl.BlockSpec(
                (B, H),
                lambda e, b, nbr, bbr: (
                    jnp.minimum(bbr[e] + jnp.maximum(
                        jnp.minimum(b, nbr[e] - 1), 0), nblk - 1), 0)),
        ),
        out_shape=jax.ShapeDtypeStruct((PADTOT, H), jnp.float32),
    )(nb, bb, sorted_h, wgu_bf, wd_bf)

    buf = pl.kernel(
        functools.partial(_sc_collect, P=P),
        mesh=mesh,
        out_type=jax.ShapeDtypeStruct((P, H), jnp.float32),
        scratch_types=[
            pltpu.VMEM((32, H), jnp.float32),
            pltpu.VMEM((32,), jnp.int32),
            pltpu.SemaphoreType.DMA,
        ],
    )(sorted_out, poscat)

    TB2 = 256
    NT2 = T // TB2
    out = pl.pallas_call(
        functools.partial(_final_body, I_sh=I_sh),
        grid=(NT2,),
        in_specs=[
            pl.BlockSpec((TB2, H), lambda t: (t, 0)),
            pl.BlockSpec((TB2, H), lambda t: (t, 0)),
            pl.BlockSpec((TB2, H), lambda t: (T // TB2 + t, 0)),
            pl.BlockSpec((TB2, E), lambda t: (t, 0)),
            pl.BlockSpec((H, 2 * I_sh), lambda t: (0, 0)),
            pl.BlockSpec((I_sh, H), lambda t: (0, 0)),
        ],
        out_specs=pl.BlockSpec((TB2, H), lambda t: (t, 0)),
        out_shape=jax.ShapeDtypeStruct((T, H), jnp.float32),
    )(hidden_states.astype(jnp.bfloat16), buf, buf, w_t8, sgu_bf, sd_bf)
    return out
